# Initial kernel scaffold; baseline (speedup 1.0000x reference)
import jax
import jax.numpy as jnp
from jax.experimental import pallas as pl


def kernel(features, indice_pairs, grid, base_weights, spline_weights):
    raise NotImplementedError("write your pallas kernel here")


# speedup vs baseline: 1.0752x; 1.0752x over previous
import functools

import jax
import jax.numpy as jnp
from jax.experimental import pallas as pl

_IN_C = 16
_OUT_C = 16
_NK = 27
_GS = 5
_SO = 3
_GEPS = 0.02
_N = 100000
_P = 50000
_SPL = _GS + _SO
_NG = _GS + 2 * _SO + 1
_CS = _IN_C * _SPL
_PC = 5000


def _bsplines_t(x_t, g):
    col = lambda j: g[:, j:j + 1]
    bases = [((x_t >= col(j)) & (x_t < col(j + 1))).astype(jnp.float32)
             for j in range(_NG - 1)]
    for k in range(1, _SO + 1):
        nxt = []
        for j in range(_NG - 1 - k):
            left = (x_t - col(j)) / (col(j + k) - col(j))
            right = (col(j + k + 1) - x_t) / (col(j + k + 1) - col(j + 1))
            nxt.append(left * bases[j] + right * bases[j + 1])
        bases = nxt
    return jnp.stack(bases, axis=1).reshape(_CS, x_t.shape[1])


def _gram_kernel(x_ref, go_ref, gn_ref, gram_ref, cross_ref):
    pc = pl.program_id(1)
    x_t = x_ref[0].T
    a_old = _bsplines_t(x_t, go_ref[0])
    a_new = _bsplines_t(x_t, gn_ref[0])
    dn = (((1,), (1,)), ((), ()))
    gram = jax.lax.dot_general(a_new, a_new, dn,
                               preferred_element_type=jnp.float32)
    cross = jax.lax.dot_general(a_new, a_old, dn,
                                preferred_element_type=jnp.float32)

    @pl.when(pc == 0)
    def _init():
        gram_ref[0] = gram
        cross_ref[0] = cross

    @pl.when(pc != 0)
    def _acc():
        gram_ref[0] += gram
        cross_ref[0] += cross


def _out_kernel(x_ref, gn_ref, w_ref, bw_ref, y_ref):
    x = x_ref[0]
    a_new = _bsplines_t(x.T, gn_ref[0])
    spline = jax.lax.dot_general(a_new, w_ref[0], (((0,), (0,)), ((), ())),
                                 preferred_element_type=jnp.float32)
    sx = x * jax.nn.sigmoid(x)
    base = jax.lax.dot_general(sx, bw_ref[0], (((1,), (1,)), ((), ())),
                               preferred_element_type=jnp.float32)
    y_ref[0] = spline + base


@jax.jit
def kernel(features, indice_pairs, grid, base_weights, spline_weights):
    inp = indice_pairs[0]
    dst = indice_pairs[1]
    x_all = features[inp]

    x_sorted = jnp.sort(x_all, axis=1)
    qidx = jnp.linspace(0.0, _P - 1, _GS + 1).astype(jnp.int32)
    grid_adaptive = x_sorted[:, qidx, :]
    x_min = x_sorted[:, 0, :]
    x_max = x_sorted[:, -1, :]
    margin = 0.01
    step = (x_max - x_min + 2 * margin) / _GS
    grid_uniform = (jnp.arange(_GS + 1, dtype=jnp.float32)[None, :, None]
                    * step[:, None, :] + (x_min - margin)[:, None, :])
    new_grid = _GEPS * grid_uniform + (1.0 - _GEPS) * grid_adaptive
    front = new_grid[:, :1, :] - step[:, None, :] * jnp.arange(
        _SO, 0, -1, dtype=jnp.float32)[None, :, None]
    back = new_grid[:, -1:, :] + step[:, None, :] * jnp.arange(
        1, _SO + 1, dtype=jnp.float32)[None, :, None]
    new_grid_full = jnp.concatenate([front, new_grid, back],
                                    axis=1).transpose(0, 2, 1)

    nchunks = _P // _PC
    gram, cross = pl.pallas_call(
        _gram_kernel,
        grid=(_NK, nchunks),
        in_specs=[
            pl.BlockSpec((1, _PC, _IN_C), lambda k, p: (k, p, 0)),
            pl.BlockSpec((1, _IN_C, _NG), lambda k, p: (k, 0, 0)),
            pl.BlockSpec((1, _IN_C, _NG), lambda k, p: (k, 0, 0)),
        ],
        out_specs=[
            pl.BlockSpec((1, _CS, _CS), lambda k, p: (k, 0, 0)),
            pl.BlockSpec((1, _CS, _CS), lambda k, p: (k, 0, 0)),
        ],
        out_shape=[
            jax.ShapeDtypeStruct((_NK, _CS, _CS), jnp.float32),
            jax.ShapeDtypeStruct((_NK, _CS, _CS), jnp.float32),
        ],
    )(x_all, grid, new_grid_full)

    cidx = jnp.arange(_IN_C)
    ata = gram.reshape(_NK, _IN_C, _SPL, _IN_C, _SPL)[
        :, cidx, :, cidx, :].transpose(1, 0, 2, 3)
    m = cross.reshape(_NK, _IN_C, _SPL, _IN_C, _SPL)[
        :, cidx, :, cidx, :].transpose(1, 0, 2, 3)
    orig = spline_weights.reshape(_NK, _OUT_C, _IN_C, _SPL).transpose(
        0, 2, 3, 1)
    atb = jnp.einsum('kcst,kcto->kcso', m, orig)
    ridge = 1e-6 * jnp.eye(_SPL, dtype=jnp.float32)
    sol = jnp.linalg.solve(ata + ridge[None, None], atb)
    w_spline = sol.reshape(_NK, _CS, _OUT_C)

    y = pl.pallas_call(
        _out_kernel,
        grid=(_NK, nchunks),
        in_specs=[
            pl.BlockSpec((1, _PC, _IN_C), lambda k, p: (k, p, 0)),
            pl.BlockSpec((1, _IN_C, _NG), lambda k, p: (k, 0, 0)),
            pl.BlockSpec((1, _CS, _OUT_C), lambda k, p: (k, 0, 0)),
            pl.BlockSpec((1, _OUT_C, _IN_C), lambda k, p: (k, 0, 0)),
        ],
        out_specs=pl.BlockSpec((1, _PC, _OUT_C), lambda k, p: (k, p, 0)),
        out_shape=jax.ShapeDtypeStruct((_NK, _P, _OUT_C), jnp.float32),
    )(x_all, new_grid_full, w_spline, base_weights)

    out = jnp.zeros((_N, _OUT_C), dtype=jnp.float32)
    return out.at[dst.reshape(-1)].add(y.reshape(-1, _OUT_C))
